# Initial kernel scaffold; baseline (speedup 1.0000x reference)
import jax
import jax.numpy as jnp
from jax.experimental import pallas as pl


def kernel(features, W_map, W_agg1, W_agg2, neigh1, neigh2):
    raise NotImplementedError("write your pallas kernel here")


# speedup vs baseline: 20.0027x; 20.0027x over previous
import functools

import jax
import jax.numpy as jnp
from jax import lax
from jax.experimental import pallas as pl
from jax.experimental.pallas import tpu as pltpu
from jax.experimental.pallas import tpu_sc as plsc

N = 100000
F = 128
D = 32
NC = 2
NS = 16
NW = NC * NS


@functools.lru_cache(maxsize=None)
def _make_gather_sum(s_fan: int, chunk: int):
    n_chunks = N // chunk
    assert n_chunks * chunk == N
    k_rows = chunk * s_fan
    assert k_rows % 8 == 0 and (chunk * D) % 8 == 0
    j_steps = (n_chunks + NW - 1) // NW

    mesh = plsc.VectorSubcoreMesh(
        core_axis_name="c", subcore_axis_name="s",
        num_cores=NC, num_subcores=NS)

    @functools.partial(
        pl.kernel,
        mesh=mesh,
        out_type=jax.ShapeDtypeStruct((N * D,), jnp.float32),
        scratch_types=[
            pltpu.VMEM((k_rows,), jnp.int32),
            pltpu.VMEM((k_rows, D), jnp.float32),
            pltpu.VMEM((chunk * D,), jnp.float32),
            pltpu.SemaphoreType.DMA,
        ],
        compiler_params=pltpu.CompilerParams(use_tc_tiling_on_sc=False),
    )
    def gsum(table_hbm, nidx_hbm, out_hbm, idx_v, rows_v, out_v, sem):
        wid = lax.axis_index("s") * NC + lax.axis_index("c")

        def step(j, carry):
            cid = wid + j * NW

            @pl.when(cid < n_chunks)
            def _():
                ioff = pl.multiple_of(cid * k_rows, 8)
                pltpu.sync_copy(nidx_hbm.at[pl.ds(ioff, k_rows)], idx_v)
                pltpu.async_copy(table_hbm.at[idx_v], rows_v, sem).wait()

                def node(c, carry2):
                    r = c * s_fan
                    a0 = rows_v[r, pl.ds(0, 16)]
                    a1 = rows_v[r, pl.ds(16, 16)]
                    for s in range(1, s_fan):
                        a0 = a0 + rows_v[r + s, pl.ds(0, 16)]
                        a1 = a1 + rows_v[r + s, pl.ds(16, 16)]
                    o = pl.multiple_of(c * D, 8)
                    out_v[pl.ds(o, 16)] = a0
                    out_v[pl.ds(o + 16, 16)] = a1
                    return carry2

                lax.fori_loop(0, chunk, node, 0)
                ooff = pl.multiple_of(cid * (chunk * D), 8)
                pltpu.sync_copy(out_v, out_hbm.at[pl.ds(ooff, chunk * D)])
            return carry

        lax.fori_loop(0, j_steps, step, 0)

    return gsum


_ROWS = 2000


def _map_body(x_ref, w_ref, o_ref):
    o_ref[...] = jnp.dot(x_ref[...], w_ref[...],
                         preferred_element_type=jnp.float32)


def _feature_map(features, wt):
    return pl.pallas_call(
        _map_body,
        grid=(N // _ROWS,),
        in_specs=[pl.BlockSpec((_ROWS, F), lambda i: (i, 0)),
                  pl.BlockSpec((F, D), lambda i: (0, 0))],
        out_specs=pl.BlockSpec((_ROWS, D), lambda i: (i, 0)),
        out_shape=jax.ShapeDtypeStruct((N, D), jnp.float32),
    )(features, wt)


def _update_body(inv, e_ref, a_ref, w_ref, o_ref):
    x = (e_ref[...] + a_ref[...]) * inv
    h = jnp.dot(x, w_ref[...], preferred_element_type=jnp.float32)
    h = jnp.maximum(h, 0.0)
    nrm = jnp.sqrt(jnp.sum(h * h, axis=1, keepdims=True))
    o_ref[...] = h / jnp.maximum(nrm, 1e-12)


def _sage_update(emb, nbr_sum, wt, inv):
    return pl.pallas_call(
        functools.partial(_update_body, inv),
        grid=(N // _ROWS,),
        in_specs=[pl.BlockSpec((_ROWS, D), lambda i: (i, 0)),
                  pl.BlockSpec((_ROWS, D), lambda i: (i, 0)),
                  pl.BlockSpec((D, D), lambda i: (0, 0))],
        out_specs=pl.BlockSpec((_ROWS, D), lambda i: (i, 0)),
        out_shape=jax.ShapeDtypeStruct((N, D), jnp.float32),
    )(emb, nbr_sum, wt)


def kernel(features, W_map, W_agg1, W_agg2, neigh1, neigh2):
    n1 = neigh1.astype(jnp.int32).reshape(-1)
    n2 = neigh2.astype(jnp.int32).reshape(-1)
    emb0 = _feature_map(features, W_map.T)
    s1 = _make_gather_sum(25, 40)(emb0, n1).reshape(N, D)
    emb1 = _sage_update(emb0, s1, W_agg1.T, 1.0 / 26.0)
    s2 = _make_gather_sum(10, 100)(emb1, n2).reshape(N, D)
    emb2 = _sage_update(emb1, s2, W_agg2.T, 1.0 / 11.0)
    return emb2


# speedup vs baseline: 25.4677x; 1.2732x over previous
import functools

import jax
import jax.numpy as jnp
from jax import lax
from jax.experimental import pallas as pl
from jax.experimental.pallas import tpu as pltpu
from jax.experimental.pallas import tpu_sc as plsc

N = 100000
F = 128
D = 32
NC = 2
NS = 16
NW = NC * NS


@functools.lru_cache(maxsize=None)
def _make_gather_sum(s_fan: int, chunk: int):
    n_chunks = N // chunk
    assert n_chunks * chunk == N
    k_rows = chunk * s_fan
    assert k_rows % 8 == 0 and (chunk * D) % 8 == 0
    j_steps = (n_chunks + NW - 1) // NW

    mesh = plsc.VectorSubcoreMesh(
        core_axis_name="c", subcore_axis_name="s",
        num_cores=NC, num_subcores=NS)

    @functools.partial(
        pl.kernel,
        mesh=mesh,
        out_type=jax.ShapeDtypeStruct((N * D,), jnp.float32),
        scratch_types=[
            pltpu.VMEM((2, k_rows), jnp.int32),
            pltpu.VMEM((2, k_rows, D), jnp.float32),
            pltpu.VMEM((chunk * D,), jnp.float32),
            pltpu.SemaphoreType.DMA,
            pltpu.SemaphoreType.DMA,
        ],
        compiler_params=pltpu.CompilerParams(use_tc_tiling_on_sc=False),
    )
    def gsum(table_hbm, nidx_hbm, out_hbm, idx_v, rows_v, out_v, sem0, sem1):
        wid = lax.axis_index("s") * NC + lax.axis_index("c")
        sems = (sem0, sem1)

        def gather_start(j, b):
            cid = wid + j * NW

            @pl.when(cid < n_chunks)
            def _():
                ioff = pl.multiple_of(cid * k_rows, 8)
                pltpu.sync_copy(nidx_hbm.at[pl.ds(ioff, k_rows)], idx_v.at[b])
                pltpu.make_async_copy(
                    table_hbm.at[idx_v.at[b]], rows_v.at[b], sems[b]).start()

        def consume(j, b):
            cid = wid + j * NW

            @pl.when(cid < n_chunks)
            def _():
                pltpu.make_async_copy(
                    table_hbm.at[idx_v.at[b]], rows_v.at[b], sems[b]).wait()

                def node(c, carry2):
                    r = c * s_fan
                    a0 = rows_v[b, r, pl.ds(0, 16)]
                    a1 = rows_v[b, r, pl.ds(16, 16)]
                    for s in range(1, s_fan):
                        a0 = a0 + rows_v[b, r + s, pl.ds(0, 16)]
                        a1 = a1 + rows_v[b, r + s, pl.ds(16, 16)]
                    o = pl.multiple_of(c * D, 8)
                    out_v[pl.ds(o, 16)] = a0
                    out_v[pl.ds(o + 16, 16)] = a1
                    return carry2

                lax.fori_loop(0, chunk, node, 0)
                ooff = pl.multiple_of(cid * (chunk * D), 8)
                pltpu.sync_copy(out_v, out_hbm.at[pl.ds(ooff, chunk * D)])

        gather_start(0, 0)

        def step(jj, carry):
            for b in (0, 1):
                j = jj * 2 + b
                gather_start(j + 1, 1 - b)
                consume(j, b)
            return carry

        lax.fori_loop(0, (j_steps + 1) // 2, step, 0)

    return gsum


_ROWS = 2000


def _map_body(x_ref, w_ref, o_ref):
    o_ref[...] = jnp.dot(x_ref[...], w_ref[...],
                         preferred_element_type=jnp.float32)


def _feature_map(features, wt):
    return pl.pallas_call(
        _map_body,
        grid=(N // _ROWS,),
        in_specs=[pl.BlockSpec((_ROWS, F), lambda i: (i, 0)),
                  pl.BlockSpec((F, D), lambda i: (0, 0))],
        out_specs=pl.BlockSpec((_ROWS, D), lambda i: (i, 0)),
        out_shape=jax.ShapeDtypeStruct((N, D), jnp.float32),
    )(features, wt)


def _update_body(inv, e_ref, a_ref, w_ref, o_ref):
    x = (e_ref[...] + a_ref[...]) * inv
    h = jnp.dot(x, w_ref[...], preferred_element_type=jnp.float32)
    h = jnp.maximum(h, 0.0)
    nrm = jnp.sqrt(jnp.sum(h * h, axis=1, keepdims=True))
    o_ref[...] = h / jnp.maximum(nrm, 1e-12)


def _sage_update(emb, nbr_sum, wt, inv):
    return pl.pallas_call(
        functools.partial(_update_body, inv),
        grid=(N // _ROWS,),
        in_specs=[pl.BlockSpec((_ROWS, D), lambda i: (i, 0)),
                  pl.BlockSpec((_ROWS, D), lambda i: (i, 0)),
                  pl.BlockSpec((D, D), lambda i: (0, 0))],
        out_specs=pl.BlockSpec((_ROWS, D), lambda i: (i, 0)),
        out_shape=jax.ShapeDtypeStruct((N, D), jnp.float32),
    )(emb, nbr_sum, wt)


def kernel(features, W_map, W_agg1, W_agg2, neigh1, neigh2):
    n1 = neigh1.astype(jnp.int32).reshape(-1)
    n2 = neigh2.astype(jnp.int32).reshape(-1)
    emb0 = _feature_map(features, W_map.T)
    s1 = _make_gather_sum(25, 40)(emb0, n1).reshape(N, D)
    emb1 = _sage_update(emb0, s1, W_agg1.T, 1.0 / 26.0)
    s2 = _make_gather_sum(10, 100)(emb1, n2).reshape(N, D)
    emb2 = _sage_update(emb1, s2, W_agg2.T, 1.0 / 11.0)
    return emb2


# speedup vs baseline: 34.1321x; 1.3402x over previous
import functools

import jax
import jax.numpy as jnp
from jax import lax
from jax.experimental import pallas as pl
from jax.experimental.pallas import tpu as pltpu
from jax.experimental.pallas import tpu_sc as plsc

N = 100000
F = 128
D = 32
NC = 2
NS = 16
NW = NC * NS


@functools.lru_cache(maxsize=None)
def _make_gather_sum(s_fan: int, chunk: int):
    n_chunks = N // chunk
    assert n_chunks * chunk == N
    k_rows = chunk * s_fan
    assert k_rows % 8 == 0 and (chunk * D) % 8 == 0
    j_steps = (n_chunks + NW - 1) // NW

    mesh = plsc.VectorSubcoreMesh(
        core_axis_name="c", subcore_axis_name="s",
        num_cores=NC, num_subcores=NS)

    @functools.partial(
        pl.kernel,
        mesh=mesh,
        out_type=jax.ShapeDtypeStruct((N * D,), jnp.float32),
        scratch_types=[
            pltpu.VMEM((2, k_rows), jnp.int32),
            pltpu.VMEM((2, k_rows, D), jnp.float32),
            pltpu.VMEM((chunk * D,), jnp.float32),
            pltpu.SemaphoreType.DMA,
            pltpu.SemaphoreType.DMA,
        ],
        compiler_params=pltpu.CompilerParams(use_tc_tiling_on_sc=False),
    )
    def gsum(table_hbm, nidx_hbm, out_hbm, idx_v, rows_v, out_v, sem0, sem1):
        wid = lax.axis_index("s") * NC + lax.axis_index("c")
        sems = (sem0, sem1)

        def gather_start(j, b):
            cid = wid + j * NW

            @pl.when(cid < n_chunks)
            def _():
                ioff = pl.multiple_of(cid * k_rows, 8)
                pltpu.sync_copy(nidx_hbm.at[pl.ds(ioff, k_rows)], idx_v.at[b])
                pltpu.make_async_copy(
                    table_hbm.at[idx_v.at[b]], rows_v.at[b], sems[b]).start()

        def consume(j, b):
            cid = wid + j * NW

            @pl.when(cid < n_chunks)
            def _():
                pltpu.make_async_copy(
                    table_hbm.at[idx_v.at[b]], rows_v.at[b], sems[b]).wait()

                def node(c, carry2):
                    r = c * s_fan
                    a0 = rows_v[b, r, pl.ds(0, 16)]
                    a1 = rows_v[b, r, pl.ds(16, 16)]
                    for s in range(1, s_fan):
                        a0 = a0 + rows_v[b, r + s, pl.ds(0, 16)]
                        a1 = a1 + rows_v[b, r + s, pl.ds(16, 16)]
                    o = pl.multiple_of(c * D, 8)
                    out_v[pl.ds(o, 16)] = a0
                    out_v[pl.ds(o + 16, 16)] = a1
                    return carry2

                lax.fori_loop(0, chunk, node, 0)
                ooff = pl.multiple_of(cid * (chunk * D), 8)
                pltpu.sync_copy(out_v, out_hbm.at[pl.ds(ooff, chunk * D)])

        gather_start(0, 0)

        def step(jj, carry):
            for b in (0, 1):
                j = jj * 2 + b
                gather_start(j + 1, 1 - b)
                consume(j, b)
            return carry

        lax.fori_loop(0, (j_steps + 1) // 2, step, 0)

    return gsum


NP = N // 4
DP = 4 * D
_PROWS = 1000


def _map_body(x_ref, w_ref, o_ref):
    o_ref[...] = jnp.dot(x_ref[...], w_ref[...],
                         preferred_element_type=jnp.float32)


def _feature_map(features, wt):
    return pl.pallas_call(
        _map_body,
        grid=(NP // _PROWS,),
        in_specs=[pl.BlockSpec((4 * _PROWS, F), lambda i: (i, 0)),
                  pl.BlockSpec((F, D), lambda i: (0, 0))],
        out_specs=pl.BlockSpec((4 * _PROWS, D), lambda i: (i, 0)),
        out_shape=jax.ShapeDtypeStruct((N, D), jnp.float32),
    )(features, wt)


def _update_body(inv, e_ref, a_ref, w_ref, s_ref, o_ref):
    x = (e_ref[...] + a_ref[...]) * inv
    h = jnp.dot(x, w_ref[...], preferred_element_type=jnp.float32)
    h = jnp.maximum(h, 0.0)
    n2 = jnp.dot(h * h, s_ref[...], preferred_element_type=jnp.float32)
    o_ref[...] = h / jnp.maximum(jnp.sqrt(n2), 1e-12)


def _sage_update_packed(embp, nbrp, w4, s4, inv):
    return pl.pallas_call(
        functools.partial(_update_body, inv),
        grid=(NP // _PROWS,),
        in_specs=[pl.BlockSpec((_PROWS, DP), lambda i: (i, 0)),
                  pl.BlockSpec((_PROWS, DP), lambda i: (i, 0)),
                  pl.BlockSpec((DP, DP), lambda i: (0, 0)),
                  pl.BlockSpec((DP, DP), lambda i: (0, 0))],
        out_specs=pl.BlockSpec((_PROWS, DP), lambda i: (i, 0)),
        out_shape=jax.ShapeDtypeStruct((NP, DP), jnp.float32),
    )(embp, nbrp, w4, s4)


def _bdiag4(w):
    z = jnp.zeros_like(w)
    return jnp.block([[w, z, z, z], [z, w, z, z], [z, z, w, z], [z, z, z, w]])


def kernel(features, W_map, W_agg1, W_agg2, neigh1, neigh2):
    n1 = neigh1.astype(jnp.int32).reshape(-1)
    n2 = neigh2.astype(jnp.int32).reshape(-1)
    w4a1 = _bdiag4(W_agg1.T)
    w4a2 = _bdiag4(W_agg2.T)
    s4 = _bdiag4(jnp.ones((D, D), jnp.float32))

    e0f = _feature_map(features, W_map.T).reshape(N * D)
    e0f = lax.optimization_barrier(e0f)
    emb0p = e0f.reshape(NP, DP)
    s1p = _make_gather_sum(25, 40)(e0f.reshape(N, D), n1).reshape(NP, DP)
    emb1p = _sage_update_packed(emb0p, s1p, w4a1, s4, 1.0 / 26.0)
    s2p = _make_gather_sum(10, 100)(emb1p.reshape(N, D), n2).reshape(NP, DP)
    emb2p = _sage_update_packed(emb1p, s2p, w4a2, s4, 1.0 / 11.0)
    return emb2p.reshape(N, D)
